# Initial kernel scaffold; baseline (speedup 1.0000x reference)
import jax
import jax.numpy as jnp
from jax.experimental import pallas as pl


def kernel(x, edge_index, edge_label_index, enc_W, enc_b, W1, b1, W2, b2, W3, b3, ep_W1, ep_b1, ep_W2, ep_b2, ep_W3, ep_b3):
    raise NotImplementedError("write your pallas kernel here")


# speedup vs baseline: 10.1099x; 10.1099x over previous
import jax
import jax.numpy as jnp
from jax import lax
from jax.experimental import pallas as pl
from jax.experimental.pallas import tpu as pltpu
from jax.experimental.pallas import tpu_sc as plsc

NN = 100000
EE = 1600000
QQ = 100000
ND = 32
HH = 128
O3 = 64
CW = 16

SB = 128
WIN = 8
EPW = SB * WIN
NWIN = 98
TPT = NWIN * EPW
EPAD = 16 * TPT
EROWS = EPAD // SB
TROWS = TPT // SB
NP = 100096
SINK = NN
RPTP = NP // 16
DEGR = 100352

QPAD = 102400
QRW = 25
QPW = QRW * SB

_MESH = plsc.VectorSubcoreMesh(
    core_axis_name="c", subcore_axis_name="s", num_cores=2, num_subcores=16)
_SC_PARAMS = pltpu.CompilerParams(use_tc_tiling_on_sc=False)

_F32 = jnp.float32


def _deg_body(dst_hbm, zeros_hbm, deg_out, idx_v, ones_v, acc, sem):
    cid = lax.axis_index("c")
    sid = lax.axis_index("s")

    @pl.when(cid == 0)
    def _():
        @pl.when(sid == 0)
        def _():
            pltpu.sync_copy(zeros_hbm, acc)
        for i in range(SB // 16):
            ones_v[pl.ds(i * 16, 16)] = jnp.full((16,), 1.0, _F32)
        plsc.subcore_barrier()

        def win(w, carry):
            row0 = sid * TROWS + w * WIN
            pltpu.sync_copy(dst_hbm.at[pl.ds(row0, WIN)], idx_v)
            ds = [pltpu.async_copy(ones_v, acc.at[idx_v.at[j]], sem, add=True)
                  for j in range(WIN)]
            for d in ds:
                d.wait()
            return carry

        lax.fori_loop(0, NWIN, win, 0)
        plsc.subcore_barrier()

        @pl.when(sid == 0)
        def _():
            pltpu.sync_copy(acc, deg_out)


_deg_call = pl.kernel(
    _deg_body,
    out_type=jax.ShapeDtypeStruct((DEGR,), _F32),
    mesh=_MESH,
    compiler_params=_SC_PARAMS,
    scratch_types=[
        pltpu.VMEM((WIN, SB), jnp.int32),
        pltpu.VMEM((SB,), _F32),
        pltpu.VMEM_SHARED((DEGR,), _F32),
        pltpu.SemaphoreType.DMA,
    ],
)


def _make_mp(nch, fout):
    rounds = nch // 2

    def body(u_ref, srcr, dstr, out_ref, uca_ref, ucb_ref, src_v, dst_v,
             rows_v, acc, gsem, ssem):
        cid = lax.axis_index("c")
        sid = lax.axis_index("s")

        def process(ch, uc_ref):
            cs = pl.ds(ch * CW, CW)
            r0 = sid * RPTP
            pltpu.sync_copy(u_ref.at[pl.ds(r0, RPTP), cs],
                            acc.at[pl.ds(r0, RPTP)])
            pltpu.sync_copy(acc.at[pl.ds(r0, RPTP)],
                            uc_ref.at[pl.ds(r0, RPTP)])
            plsc.subcore_barrier()

            def win(w, carry):
                row0 = sid * TROWS + w * WIN
                pltpu.sync_copy(srcr.at[pl.ds(row0, WIN)], src_v)
                pltpu.sync_copy(dstr.at[pl.ds(row0, WIN)], dst_v)
                gd = [pltpu.async_copy(uc_ref.at[src_v.at[j]],
                                       rows_v.at[pl.ds(j * SB, SB)], gsem)
                      for j in range(WIN)]
                for d in gd:
                    d.wait()
                sd = [pltpu.async_copy(rows_v.at[pl.ds(j * SB, SB)],
                                       acc.at[dst_v.at[j]], ssem, add=True)
                      for j in range(WIN)]
                for d in sd:
                    d.wait()
                return carry

            lax.fori_loop(0, NWIN, win, 0)
            plsc.subcore_barrier()
            pltpu.sync_copy(acc.at[pl.ds(r0, RPTP)],
                            out_ref.at[pl.ds(r0, RPTP), cs])
            plsc.subcore_barrier()

        for r in range(rounds):
            @pl.when(cid == 0)
            def _(r=r):
                process(r, uca_ref)

            @pl.when(cid == 1)
            def _(r=r):
                process(rounds + r, ucb_ref)

    return pl.kernel(
        body,
        out_type=[jax.ShapeDtypeStruct((NP, fout), _F32),
                  jax.ShapeDtypeStruct((NP, CW), _F32),
                  jax.ShapeDtypeStruct((NP, CW), _F32)],
        mesh=_MESH,
        compiler_params=_SC_PARAMS,
        scratch_types=[
            pltpu.VMEM((WIN, SB), jnp.int32),
            pltpu.VMEM((WIN, SB), jnp.int32),
            pltpu.VMEM((EPW, CW), _F32),
            pltpu.VMEM_SHARED((NP, CW), _F32),
            pltpu.SemaphoreType.DMA,
            pltpu.SemaphoreType.DMA,
        ],
    )


_mp8 = _make_mp(8, HH)
_mp4 = _make_mp(4, O3)


_QPASS = ((0, 12), (12, 13))


def _gather_body(s3_ref, dinv_hbm, q0_hbm, q1_hbm, ef_ref, odv0, odv1,
                 qv, buf, dbuf, sem):
    cid = lax.axis_index("c")
    sid = lax.axis_index("s")
    wid = sid * 2 + cid
    base_row = wid * QRW
    obase = wid * QPW

    for q_hbm, coff, odv in ((q0_hbm, 0, odv0), (q1_hbm, O3, odv1)):
        pltpu.sync_copy(q_hbm.at[pl.ds(base_row, QRW)], qv)
        for j0, jn in _QPASS:
            gd = [pltpu.async_copy(s3_ref.at[qv.at[j0 + j]],
                                   buf.at[pl.ds(j * SB, SB)], sem)
                  for j in range(jn)]
            for d in gd:
                d.wait()
            pltpu.sync_copy(
                buf.at[pl.ds(0, jn * SB)],
                ef_ref.at[pl.ds(obase + j0 * SB, jn * SB), pl.ds(coff, O3)])
        dd = [pltpu.async_copy(dinv_hbm.at[qv.at[j]],
                               dbuf.at[pl.ds(j * SB, SB)], sem)
              for j in range(QRW)]
        for d in dd:
            d.wait()
        pltpu.sync_copy(dbuf, odv.at[pl.ds(obase, QPW)])


_gather_call = pl.kernel(
    _gather_body,
    out_type=[jax.ShapeDtypeStruct((QPAD, HH), _F32),
              jax.ShapeDtypeStruct((QPAD,), _F32),
              jax.ShapeDtypeStruct((QPAD,), _F32)],
    mesh=_MESH,
    compiler_params=_SC_PARAMS,
    scratch_types=[
        pltpu.VMEM((QRW, SB), jnp.int32),
        pltpu.VMEM((13 * SB, O3), _F32),
        pltpu.VMEM((QPW,), _F32),
        pltpu.SemaphoreType.DMA,
    ],
)


_BM = 3128
_BMQ = 3200


def _tca_body(x_ref, ew_ref, eb_ref, deg_ref, w1_ref, dv_ref, u_ref):
    h = jnp.dot(x_ref[...], ew_ref[...], preferred_element_type=_F32)
    h = jnp.maximum(h + eb_ref[...], 0.0)
    dv = lax.rsqrt(deg_ref[...] + 1.0)
    u_ref[...] = jnp.dot(h, w1_ref[...], preferred_element_type=_F32) * dv
    dv_ref[...] = dv


def _tca(x, enc_w, enc_b, deg2d, w1):
    return pl.pallas_call(
        _tca_body,
        grid=(NP // _BM,),
        in_specs=[
            pl.BlockSpec((_BM, ND), lambda i: (i, 0)),
            pl.BlockSpec((ND, HH), lambda i: (0, 0)),
            pl.BlockSpec((1, HH), lambda i: (0, 0)),
            pl.BlockSpec((_BM, 1), lambda i: (i, 0)),
            pl.BlockSpec((HH, HH), lambda i: (0, 0)),
        ],
        out_specs=[pl.BlockSpec((_BM, 1), lambda i: (i, 0)),
                   pl.BlockSpec((_BM, HH), lambda i: (i, 0))],
        out_shape=[jax.ShapeDtypeStruct((NP, 1), _F32),
                   jax.ShapeDtypeStruct((NP, HH), _F32)],
    )(x, enc_w, enc_b, deg2d, w1)


def _transform_body(s_ref, dv_ref, b_ref, w_ref, u_ref):
    dv = dv_ref[...]
    h = jnp.maximum(s_ref[...] * dv + b_ref[...], 0.0)
    u_ref[...] = jnp.dot(h, w_ref[...], preferred_element_type=_F32) * dv


def _transform(s, dinv2d, b, w):
    return pl.pallas_call(
        _transform_body,
        grid=(NP // _BM,),
        in_specs=[
            pl.BlockSpec((_BM, HH), lambda i: (i, 0)),
            pl.BlockSpec((_BM, 1), lambda i: (i, 0)),
            pl.BlockSpec((1, HH), lambda i: (0, 0)),
            pl.BlockSpec((HH, HH), lambda i: (0, 0)),
        ],
        out_specs=pl.BlockSpec((_BM, HH), lambda i: (i, 0)),
        out_shape=jax.ShapeDtypeStruct((NP, HH), _F32),
    )(s, dinv2d, b, w)


def _tcd_body(ef_ref, dv0_ref, dv1_ref, b3_ref, w1_ref, b1_ref, w2_ref,
              b2_ref, w3_ref, b3e_ref, out_ref):
    e = ef_ref[...]
    s_n = e[:, 0:O3] * dv0_ref[...] + b3_ref[...]
    d_n = e[:, O3:HH] * dv1_ref[...] + b3_ref[...]
    ef = jnp.concatenate([s_n, d_n], axis=1)
    p = jnp.dot(ef, w1_ref[...], preferred_element_type=_F32) + b1_ref[...]
    p = jnp.maximum(p, 0.0)
    p = jnp.dot(p, w2_ref[...], preferred_element_type=_F32) + b2_ref[...]
    p = jnp.maximum(p, 0.0)
    z = jnp.dot(p, w3_ref[...], preferred_element_type=_F32) + b3e_ref[...]
    out_ref[...] = jax.nn.sigmoid(z)


def _tcd(ef, dv0, dv1, b3, ep_w1, ep_b1, ep_w2, ep_b2, ep_w3, ep_b3):
    return pl.pallas_call(
        _tcd_body,
        grid=(QPAD // _BMQ,),
        in_specs=[
            pl.BlockSpec((_BMQ, HH), lambda i: (i, 0)),
            pl.BlockSpec((_BMQ, 1), lambda i: (i, 0)),
            pl.BlockSpec((_BMQ, 1), lambda i: (i, 0)),
            pl.BlockSpec((1, O3), lambda i: (0, 0)),
            pl.BlockSpec((HH, HH), lambda i: (0, 0)),
            pl.BlockSpec((1, HH), lambda i: (0, 0)),
            pl.BlockSpec((HH, O3), lambda i: (0, 0)),
            pl.BlockSpec((1, O3), lambda i: (0, 0)),
            pl.BlockSpec((O3, 1), lambda i: (0, 0)),
            pl.BlockSpec((1, 1), lambda i: (0, 0)),
        ],
        out_specs=pl.BlockSpec((_BMQ, 1), lambda i: (i, 0)),
        out_shape=jax.ShapeDtypeStruct((QPAD, 1), _F32),
    )(ef, dv0, dv1, b3, ep_w1, ep_b1, ep_w2, ep_b2, ep_w3, ep_b3)


def kernel(x, edge_index, edge_label_index, enc_W, enc_b, W1, b1, W2, b2,
           W3, b3, ep_W1, ep_b1, ep_W2, ep_b2, ep_W3, ep_b3):
    src = edge_index[0]
    dst = edge_index[1]
    src_p = jnp.concatenate(
        [src, jnp.zeros((EPAD - EE,), jnp.int32)]).reshape(EROWS, SB)
    dst_p = jnp.concatenate(
        [dst, jnp.full((EPAD - EE,), SINK, jnp.int32)]).reshape(EROWS, SB)
    zeros = jnp.zeros((DEGR,), _F32)

    deg = _deg_call(dst_p, zeros)
    x_p = jnp.concatenate([x, jnp.zeros((NP - NN, ND), _F32)])
    dinv2d, u1 = _tca(x_p, enc_W, enc_b.reshape(1, HH),
                      deg[:NP].reshape(NP, 1), W1)

    s1, _, _ = _mp8(u1, src_p, dst_p)
    u2 = _transform(s1, dinv2d, b1.reshape(1, HH), W2)
    s2, _, _ = _mp8(u2, src_p, dst_p)
    w3_pad = jnp.concatenate([W3, jnp.zeros((HH, HH - O3), _F32)], axis=1)
    u3 = _transform(s2, dinv2d, b2.reshape(1, HH), w3_pad)
    s3, _, _ = _mp4(u3, src_p, dst_p)

    qpad = jnp.zeros((QPAD - QQ,), jnp.int32)
    q0 = jnp.concatenate([edge_label_index[0], qpad]).reshape(QPAD // SB, SB)
    q1 = jnp.concatenate([edge_label_index[1], qpad]).reshape(QPAD // SB, SB)
    ef, dv0, dv1 = _gather_call(s3, dinv2d.reshape(NP), q0, q1)

    p = _tcd(ef, dv0.reshape(QPAD, 1), dv1.reshape(QPAD, 1),
             b3.reshape(1, O3), ep_W1, ep_b1.reshape(1, HH), ep_W2,
             ep_b2.reshape(1, O3), ep_W3, ep_b3.reshape(1, 1))
    return p[:QQ, 0]


# speedup vs baseline: 12.3736x; 1.2239x over previous
import jax
import jax.numpy as jnp
from jax import lax
from jax.experimental import pallas as pl
from jax.experimental.pallas import tpu as pltpu
from jax.experimental.pallas import tpu_sc as plsc

NN = 100000
EE = 1600000
QQ = 100000
ND = 32
HH = 128
O3 = 64
CW = 16

SB = 128
WIN = 4
EPW = SB * WIN
NWIN = 196
TPT = NWIN * EPW
EPAD = 16 * TPT
EROWS = EPAD // SB + WIN
TROWS = TPT // SB
NP = 100096
SINK = NN
RPTP = NP // 16
DEGR = 100352

QPAD = 102400
QRW = 25
QPW = QRW * SB

_MESH = plsc.VectorSubcoreMesh(
    core_axis_name="c", subcore_axis_name="s", num_cores=2, num_subcores=16)
_SC_PARAMS = pltpu.CompilerParams(use_tc_tiling_on_sc=False)

_F32 = jnp.float32


def _deg_body(dst_hbm, zeros_hbm, deg_out, idx_v, ones_v, acc, sem):
    cid = lax.axis_index("c")
    sid = lax.axis_index("s")

    @pl.when(cid == 0)
    def _():
        @pl.when(sid == 0)
        def _():
            pltpu.sync_copy(zeros_hbm, acc)
        for i in range(SB // 16):
            ones_v[pl.ds(i * 16, 16)] = jnp.full((16,), 1.0, _F32)
        plsc.subcore_barrier()

        def win(w, carry):
            row0 = sid * TROWS + w * WIN
            pltpu.sync_copy(dst_hbm.at[pl.ds(row0, WIN)], idx_v)
            ds = [pltpu.async_copy(ones_v, acc.at[idx_v.at[j]], sem, add=True)
                  for j in range(WIN)]
            for d in ds:
                d.wait()
            return carry

        lax.fori_loop(0, NWIN, win, 0)
        plsc.subcore_barrier()

        @pl.when(sid == 0)
        def _():
            pltpu.sync_copy(acc, deg_out)


_deg_call = pl.kernel(
    _deg_body,
    out_type=jax.ShapeDtypeStruct((DEGR,), _F32),
    mesh=_MESH,
    compiler_params=_SC_PARAMS,
    scratch_types=[
        pltpu.VMEM((WIN, SB), jnp.int32),
        pltpu.VMEM((SB,), _F32),
        pltpu.VMEM_SHARED((DEGR,), _F32),
        pltpu.SemaphoreType.DMA,
    ],
)


def _make_mp(nch, fout):
    rounds = nch // 2

    def body(u_ref, e2r, out_ref, uca_ref, ucb_ref, ev0, ev1, rows0, rows1,
             acc, gsem, ssem):
        cid = lax.axis_index("c")
        sid = lax.axis_index("s")

        def process(ch, uc_ref):
            cs = pl.ds(ch * CW, CW)
            r0 = sid * RPTP
            base_row = sid * TROWS
            pltpu.sync_copy(u_ref.at[pl.ds(r0, RPTP), cs],
                            acc.at[pl.ds(r0, RPTP)])
            pltpu.sync_copy(acc.at[pl.ds(r0, RPTP)],
                            uc_ref.at[pl.ds(r0, RPTP)])
            plsc.subcore_barrier()

            def gfire(ev, rows):
                for j in range(WIN):
                    pltpu.async_copy(uc_ref.at[ev.at[j, 0]],
                                     rows.at[pl.ds(j * SB, SB)], gsem)

            def gdrain(ev, rows):
                for j in range(WIN):
                    pltpu.make_async_copy(uc_ref.at[ev.at[j, 0]],
                                          rows.at[pl.ds(j * SB, SB)],
                                          gsem).wait()

            def sflush(ev, rows):
                sd = [pltpu.async_copy(rows.at[pl.ds(j * SB, SB)],
                                      acc.at[ev.at[j, 1]], ssem, add=True)
                      for j in range(WIN)]
                for d in sd:
                    d.wait()

            pltpu.sync_copy(e2r.at[pl.ds(base_row, WIN)], ev0)
            gfire(ev0, rows0)

            def pair(g, carry):
                w0row = base_row + (2 * g) * WIN
                pltpu.sync_copy(e2r.at[pl.ds(w0row + WIN, WIN)], ev1)
                gdrain(ev0, rows0)
                gfire(ev1, rows1)
                sflush(ev0, rows0)
                pltpu.sync_copy(e2r.at[pl.ds(w0row + 2 * WIN, WIN)], ev0)
                gdrain(ev1, rows1)
                gfire(ev0, rows0)
                sflush(ev1, rows1)
                return carry

            lax.fori_loop(0, NWIN // 2, pair, 0)
            gdrain(ev0, rows0)
            plsc.subcore_barrier()
            pltpu.sync_copy(acc.at[pl.ds(r0, RPTP)],
                            out_ref.at[pl.ds(r0, RPTP), cs])
            plsc.subcore_barrier()

        for r in range(rounds):
            @pl.when(cid == 0)
            def _(r=r):
                process(r, uca_ref)

            @pl.when(cid == 1)
            def _(r=r):
                process(rounds + r, ucb_ref)

    return pl.kernel(
        body,
        out_type=[jax.ShapeDtypeStruct((NP, fout), _F32),
                  jax.ShapeDtypeStruct((NP, CW), _F32),
                  jax.ShapeDtypeStruct((NP, CW), _F32)],
        mesh=_MESH,
        compiler_params=_SC_PARAMS,
        scratch_types=[
            pltpu.VMEM((WIN, 2, SB), jnp.int32),
            pltpu.VMEM((WIN, 2, SB), jnp.int32),
            pltpu.VMEM((EPW, CW), _F32),
            pltpu.VMEM((EPW, CW), _F32),
            pltpu.VMEM_SHARED((NP, CW), _F32),
            pltpu.SemaphoreType.DMA,
            pltpu.SemaphoreType.DMA,
        ],
    )


_mp8 = _make_mp(8, HH)
_mp4 = _make_mp(4, O3)


_QPASS = ((0, 12), (12, 13))


def _gather_body(s3_ref, dinv_hbm, q0_hbm, q1_hbm, ef_ref, odv0, odv1,
                 qv, buf, dbuf, sem):
    cid = lax.axis_index("c")
    sid = lax.axis_index("s")
    wid = sid * 2 + cid
    base_row = wid * QRW
    obase = wid * QPW

    for q_hbm, coff, odv in ((q0_hbm, 0, odv0), (q1_hbm, O3, odv1)):
        pltpu.sync_copy(q_hbm.at[pl.ds(base_row, QRW)], qv)
        for j0, jn in _QPASS:
            gd = [pltpu.async_copy(s3_ref.at[qv.at[j0 + j]],
                                   buf.at[pl.ds(j * SB, SB)], sem)
                  for j in range(jn)]
            for d in gd:
                d.wait()
            pltpu.sync_copy(
                buf.at[pl.ds(0, jn * SB)],
                ef_ref.at[pl.ds(obase + j0 * SB, jn * SB), pl.ds(coff, O3)])
        dd = [pltpu.async_copy(dinv_hbm.at[qv.at[j]],
                               dbuf.at[pl.ds(j * SB, SB)], sem)
              for j in range(QRW)]
        for d in dd:
            d.wait()
        pltpu.sync_copy(dbuf, odv.at[pl.ds(obase, QPW)])


_gather_call = pl.kernel(
    _gather_body,
    out_type=[jax.ShapeDtypeStruct((QPAD, HH), _F32),
              jax.ShapeDtypeStruct((QPAD,), _F32),
              jax.ShapeDtypeStruct((QPAD,), _F32)],
    mesh=_MESH,
    compiler_params=_SC_PARAMS,
    scratch_types=[
        pltpu.VMEM((QRW, SB), jnp.int32),
        pltpu.VMEM((13 * SB, O3), _F32),
        pltpu.VMEM((QPW,), _F32),
        pltpu.SemaphoreType.DMA,
    ],
)


_BM = 3128
_BMQ = 3200


def _tca_body(x_ref, ew_ref, eb_ref, deg_ref, w1_ref, dv_ref, u_ref):
    h = jnp.dot(x_ref[...], ew_ref[...], preferred_element_type=_F32)
    h = jnp.maximum(h + eb_ref[...], 0.0)
    dv = lax.rsqrt(deg_ref[...] + 1.0)
    u_ref[...] = jnp.dot(h, w1_ref[...], preferred_element_type=_F32) * dv
    dv_ref[...] = dv


def _tca(x, enc_w, enc_b, deg2d, w1):
    return pl.pallas_call(
        _tca_body,
        grid=(NP // _BM,),
        in_specs=[
            pl.BlockSpec((_BM, ND), lambda i: (i, 0)),
            pl.BlockSpec((ND, HH), lambda i: (0, 0)),
            pl.BlockSpec((1, HH), lambda i: (0, 0)),
            pl.BlockSpec((_BM, 1), lambda i: (i, 0)),
            pl.BlockSpec((HH, HH), lambda i: (0, 0)),
        ],
        out_specs=[pl.BlockSpec((_BM, 1), lambda i: (i, 0)),
                   pl.BlockSpec((_BM, HH), lambda i: (i, 0))],
        out_shape=[jax.ShapeDtypeStruct((NP, 1), _F32),
                   jax.ShapeDtypeStruct((NP, HH), _F32)],
    )(x, enc_w, enc_b, deg2d, w1)


def _transform_body(s_ref, dv_ref, b_ref, w_ref, u_ref):
    dv = dv_ref[...]
    h = jnp.maximum(s_ref[...] * dv + b_ref[...], 0.0)
    u_ref[...] = jnp.dot(h, w_ref[...], preferred_element_type=_F32) * dv


def _transform(s, dinv2d, b, w):
    return pl.pallas_call(
        _transform_body,
        grid=(NP // _BM,),
        in_specs=[
            pl.BlockSpec((_BM, HH), lambda i: (i, 0)),
            pl.BlockSpec((_BM, 1), lambda i: (i, 0)),
            pl.BlockSpec((1, HH), lambda i: (0, 0)),
            pl.BlockSpec((HH, HH), lambda i: (0, 0)),
        ],
        out_specs=pl.BlockSpec((_BM, HH), lambda i: (i, 0)),
        out_shape=jax.ShapeDtypeStruct((NP, HH), _F32),
    )(s, dinv2d, b, w)


def _tcd_body(ef_ref, dv0_ref, dv1_ref, b3_ref, w1_ref, b1_ref, w2_ref,
              b2_ref, w3_ref, b3e_ref, out_ref):
    e = ef_ref[...]
    s_n = e[:, 0:O3] * dv0_ref[...] + b3_ref[...]
    d_n = e[:, O3:HH] * dv1_ref[...] + b3_ref[...]
    ef = jnp.concatenate([s_n, d_n], axis=1)
    p = jnp.dot(ef, w1_ref[...], preferred_element_type=_F32) + b1_ref[...]
    p = jnp.maximum(p, 0.0)
    p = jnp.dot(p, w2_ref[...], preferred_element_type=_F32) + b2_ref[...]
    p = jnp.maximum(p, 0.0)
    z = jnp.dot(p, w3_ref[...], preferred_element_type=_F32) + b3e_ref[...]
    out_ref[...] = jax.nn.sigmoid(z)


def _tcd(ef, dv0, dv1, b3, ep_w1, ep_b1, ep_w2, ep_b2, ep_w3, ep_b3):
    return pl.pallas_call(
        _tcd_body,
        grid=(QPAD // _BMQ,),
        in_specs=[
            pl.BlockSpec((_BMQ, HH), lambda i: (i, 0)),
            pl.BlockSpec((_BMQ, 1), lambda i: (i, 0)),
            pl.BlockSpec((_BMQ, 1), lambda i: (i, 0)),
            pl.BlockSpec((1, O3), lambda i: (0, 0)),
            pl.BlockSpec((HH, HH), lambda i: (0, 0)),
            pl.BlockSpec((1, HH), lambda i: (0, 0)),
            pl.BlockSpec((HH, O3), lambda i: (0, 0)),
            pl.BlockSpec((1, O3), lambda i: (0, 0)),
            pl.BlockSpec((O3, 1), lambda i: (0, 0)),
            pl.BlockSpec((1, 1), lambda i: (0, 0)),
        ],
        out_specs=pl.BlockSpec((_BMQ, 1), lambda i: (i, 0)),
        out_shape=jax.ShapeDtypeStruct((QPAD, 1), _F32),
    )(ef, dv0, dv1, b3, ep_w1, ep_b1, ep_w2, ep_b2, ep_w3, ep_b3)


def kernel(x, edge_index, edge_label_index, enc_W, enc_b, W1, b1, W2, b2,
           W3, b3, ep_W1, ep_b1, ep_W2, ep_b2, ep_W3, ep_b3):
    src = edge_index[0]
    dst = edge_index[1]
    npad = EROWS * SB - EE
    src_p = jnp.concatenate(
        [src, jnp.zeros((npad,), jnp.int32)]).reshape(EROWS, SB)
    dst_p = jnp.concatenate(
        [dst, jnp.full((npad,), SINK, jnp.int32)]).reshape(EROWS, SB)
    e2 = jnp.stack([src_p, dst_p], axis=1)
    zeros = jnp.zeros((DEGR,), _F32)

    deg = _deg_call(dst_p, zeros)
    x_p = jnp.concatenate([x, jnp.zeros((NP - NN, ND), _F32)])
    dinv2d, u1 = _tca(x_p, enc_W, enc_b.reshape(1, HH),
                      deg[:NP].reshape(NP, 1), W1)

    s1, _, _ = _mp8(u1, e2)
    u2 = _transform(s1, dinv2d, b1.reshape(1, HH), W2)
    s2, _, _ = _mp8(u2, e2)
    w3_pad = jnp.concatenate([W3, jnp.zeros((HH, HH - O3), _F32)], axis=1)
    u3 = _transform(s2, dinv2d, b2.reshape(1, HH), w3_pad)
    s3, _, _ = _mp4(u3, e2)

    qpad = jnp.zeros((QPAD - QQ,), jnp.int32)
    q0 = jnp.concatenate([edge_label_index[0], qpad]).reshape(QPAD // SB, SB)
    q1 = jnp.concatenate([edge_label_index[1], qpad]).reshape(QPAD // SB, SB)
    ef, dv0, dv1 = _gather_call(s3, dinv2d.reshape(NP), q0, q1)

    p = _tcd(ef, dv0.reshape(QPAD, 1), dv1.reshape(QPAD, 1),
             b3.reshape(1, O3), ep_W1, ep_b1.reshape(1, HH), ep_W2,
             ep_b2.reshape(1, O3), ep_W3, ep_b3.reshape(1, 1))
    return p[:QQ, 0]
